# baseline (speedup 1.0000x reference)
import functools

import jax
import jax.numpy as jnp
from jax import lax
from jax.experimental import pallas as pl
from jax.experimental.pallas import tpu as pltpu
from jax.experimental.pallas import tpu_sc as plsc

B = 16
CT = 32
TT = 32
R = 31
TILE = 8 * 128
SLAB = TT * TILE
HCT = 16
RUN = 2 * TILE
NW = 32
UNITS = B * R * (CT // HCT)
UPW = UNITS // NW
NBUF = 3


def _sc_chunk(x_lin):
    mesh = plsc.VectorSubcoreMesh(core_axis_name="c", subcore_axis_name="s")

    @functools.partial(
        pl.kernel,
        out_type=jax.ShapeDtypeStruct((UNITS * HCT * RUN,), jnp.float32),
        mesh=mesh,
        compiler_params=pltpu.CompilerParams(needs_layout_passes=False),
        scratch_types=(
            [pltpu.VMEM((HCT * RUN,), jnp.float32)] * NBUF
            + [pltpu.SemaphoreType.DMA] * NBUF
            + [pltpu.SemaphoreType.DMA] * NBUF
        ),
    )
    def k(x_hbm, out_hbm, buf0, buf1, buf2, si0, si1, si2, so0, so1, so2):
        wid = lax.axis_index("s") * 2 + lax.axis_index("c")
        bufs, sis, sos = (buf0, buf1, buf2), (si0, si1, si2), (so0, so1, so2)

        def unit(j):
            u = wid + j * NW
            br, h = u >> 1, u & 1
            b, r = br // R, br % R
            return b, r, h, br

        def in_dmas(j, p):
            b, r, h, _ = unit(j)
            row0 = b * CT + h * HCT
            return [
                pltpu.make_async_copy(
                    x_hbm.at[pl.ds((row0 + i) * SLAB + r * TILE, RUN)],
                    bufs[p].at[pl.ds(i * RUN, RUN)], sis[p])
                for i in range(HCT)
            ]

        def out_dma(j, p):
            _, _, h, br = unit(j)
            return pltpu.make_async_copy(
                bufs[p],
                out_hbm.at[pl.ds((br * 2 + h) * HCT * RUN, HCT * RUN)],
                sos[p])

        def start_in(j, p):
            for d in in_dmas(j, p):
                d.start()

        def wait_in(j, p):
            for d in in_dmas(j, p):
                d.wait()

        start_in(0, 0)
        start_in(1, 1)
        for j in range(UPW):
            p = j % NBUF
            wait_in(j, p)
            out_dma(j, p).start()
            if j + 2 < UPW:
                q = (j + 2) % NBUF
                if j >= 1:
                    out_dma(j - 1, q).wait()
                start_in(j + 2, q)
        out_dma(UPW - 2, (UPW - 2) % NBUF).wait()
        out_dma(UPW - 1, (UPW - 1) % NBUF).wait()

    return k(x_lin)


def kernel(x):
    x_lin = x.reshape(B, CT, 8, TT, 128).transpose(0, 1, 3, 2, 4).reshape(-1)
    out_lin = _sc_chunk(x_lin)
    out = (out_lin.reshape(B, R, CT, 2, 8, 128)
           .transpose(0, 2, 4, 3, 5, 1)
           .reshape(16, 256, 256, 31))
    return out


# speedup vs baseline: 1.2219x; 1.2219x over previous
import functools

import jax
import jax.numpy as jnp
from jax import lax
from jax.experimental import pallas as pl
from jax.experimental.pallas import tpu as pltpu
from jax.experimental.pallas import tpu_sc as plsc

B = 16
CT = 32
TT = 32
R = 31
TILE = 8 * 128
SLAB = TT * TILE
OSLAB = 2 * TILE
NW = 32
SPW = (B * CT) // NW
NBUF = 3


def _sc_chunk(x_lin):
    mesh = plsc.VectorSubcoreMesh(core_axis_name="c", subcore_axis_name="s")

    @functools.partial(
        pl.kernel,
        out_type=jax.ShapeDtypeStruct((B * R * CT * OSLAB,), jnp.float32),
        mesh=mesh,
        compiler_params=pltpu.CompilerParams(needs_layout_passes=False),
        scratch_types=(
            [pltpu.VMEM((SLAB,), jnp.float32)] * NBUF
            + [pltpu.SemaphoreType.DMA] * NBUF
            + [pltpu.SemaphoreType.DMA] * NBUF
        ),
    )
    def k(x_hbm, out_hbm, buf0, buf1, buf2, si0, si1, si2, so0, so1, so2):
        wid = lax.axis_index("s") * 2 + lax.axis_index("c")
        s0 = wid * SPW
        bufs, sis, sos = (buf0, buf1, buf2), (si0, si1, si2), (so0, so1, so2)

        def in_dma(i, p):
            return pltpu.make_async_copy(
                x_hbm.at[pl.ds((s0 + i) * SLAB, SLAB)], bufs[p], sis[p])

        def out_dma(i, r, p):
            s = s0 + i
            b, ct = s >> 5, s & 31
            off = ((b * R + r) * CT + ct) * OSLAB
            return pltpu.make_async_copy(
                bufs[p].at[pl.ds(r * TILE, OSLAB)],
                out_hbm.at[pl.ds(off, OSLAB)], sos[p])

        in_dma(0, 0).start()
        in_dma(1, 1).start()
        for i in range(SPW):
            p = i % NBUF
            in_dma(i, p).wait()
            for r in range(R):
                out_dma(i, r, p).start()
            if i + 2 < SPW:
                q = (i + 2) % NBUF
                if i >= 1:
                    for r in range(R):
                        out_dma(i - 1, r, q).wait()
                in_dma(i + 2, q).start()
        for r in range(R):
            out_dma(SPW - 2, r, (SPW - 2) % NBUF).wait()
        for r in range(R):
            out_dma(SPW - 1, r, (SPW - 1) % NBUF).wait()

    return k(x_lin)


def kernel(x):
    x_lin = x.reshape(B, CT, 8, TT, 128).transpose(0, 1, 3, 2, 4).reshape(-1)
    out_lin = _sc_chunk(x_lin)
    out = (out_lin.reshape(B, R, CT, 2, 8, 128)
           .transpose(0, 2, 4, 3, 5, 1)
           .reshape(16, 256, 256, 31))
    return out


# speedup vs baseline: 1.2758x; 1.0441x over previous
import functools

import jax
import jax.numpy as jnp
from jax import lax
from jax.experimental import pallas as pl
from jax.experimental.pallas import tpu as pltpu
from jax.experimental.pallas import tpu_sc as plsc

B = 16
CT = 32
TT = 32
R = 31
TILE = 8 * 128
SLAB = TT * TILE
OSLAB = 2 * TILE
NW = 32
SPW = (B * CT) // NW


def _sc_chunk(x_lin):
    mesh = plsc.VectorSubcoreMesh(core_axis_name="c", subcore_axis_name="s")

    @functools.partial(
        pl.kernel,
        out_type=jax.ShapeDtypeStruct((B * R * CT * OSLAB,), jnp.float32),
        mesh=mesh,
        compiler_params=pltpu.CompilerParams(needs_layout_passes=False),
        scratch_types=[
            pltpu.VMEM((SLAB,), jnp.float32),
            pltpu.VMEM((SLAB,), jnp.float32),
            pltpu.SemaphoreType.DMA,
            pltpu.SemaphoreType.DMA,
            pltpu.SemaphoreType.DMA,
            pltpu.SemaphoreType.DMA,
        ],
    )
    def k(x_hbm, out_hbm, buf0, buf1, si0, si1, so0, so1):
        wid = lax.axis_index("s") * 2 + lax.axis_index("c")
        bufs, sis, sos = (buf0, buf1), (si0, si1), (so0, so1)

        def in_dma(i, p):
            s = wid + i * NW
            return pltpu.make_async_copy(
                x_hbm.at[pl.ds(s * SLAB, SLAB)], bufs[p], sis[p])

        def out_dma(i, r, p):
            s = wid + i * NW
            b, ct = s >> 5, s & 31
            off = ((b * R + r) * CT + ct) * OSLAB
            return pltpu.make_async_copy(
                bufs[p].at[pl.ds(r * TILE, OSLAB)],
                out_hbm.at[pl.ds(off, OSLAB)], sos[p])

        in_dma(0, 0).start()

        def step(i, p):
            in_dma(i, p).wait()
            for r in range(R):
                out_dma(i, r, p).start()

            @pl.when(i + 1 < SPW)
            def _():
                @pl.when(i >= 1)
                def _():
                    for r in range(R):
                        out_dma(i - 1, r, 1 - p).wait()

                in_dma(i + 1, 1 - p).start()

        def pair(k2, _):
            step(k2 * 2, 0)
            step(k2 * 2 + 1, 1)
            return 0

        lax.fori_loop(0, SPW // 2, pair, 0)
        for r in range(R):
            out_dma(SPW - 2, r, 0).wait()
        for r in range(R):
            out_dma(SPW - 1, r, 1).wait()

    return k(x_lin)


def kernel(x):
    x_lin = x.reshape(B, CT, 8, TT, 128).transpose(0, 1, 3, 2, 4).reshape(-1)
    out_lin = _sc_chunk(x_lin)
    out = (out_lin.reshape(B, R, CT, 2, 8, 128)
           .transpose(0, 2, 4, 3, 5, 1)
           .reshape(16, 256, 256, 31))
    return out


# speedup vs baseline: 1.2821x; 1.0050x over previous
import functools

import jax
import jax.numpy as jnp
from jax import lax
from jax.experimental import pallas as pl
from jax.experimental.pallas import tpu as pltpu
from jax.experimental.pallas import tpu_sc as plsc

B = 16
CT = 32
TT = 32
R = 31
TILE = 8 * 128
SLAB = TT * TILE
OSLAB = 2 * TILE
NW = 32
SPW = (B * CT) // NW


def _sc_chunk(x_lin):
    mesh = plsc.VectorSubcoreMesh(core_axis_name="c", subcore_axis_name="s")

    @functools.partial(
        pl.kernel,
        out_type=jax.ShapeDtypeStruct((B * R * CT * OSLAB,), jnp.float32),
        mesh=mesh,
        compiler_params=pltpu.CompilerParams(
            needs_layout_passes=False,
            disable_bounds_checks=True,
            disable_semaphore_checks=True,
        ),
        scratch_types=[
            pltpu.VMEM((SLAB,), jnp.float32),
            pltpu.VMEM((SLAB,), jnp.float32),
            pltpu.SemaphoreType.DMA,
            pltpu.SemaphoreType.DMA,
            pltpu.SemaphoreType.DMA,
            pltpu.SemaphoreType.DMA,
        ],
    )
    def k(x_hbm, out_hbm, buf0, buf1, si0, si1, so0, so1):
        wid = lax.axis_index("s") * 2 + lax.axis_index("c")
        bufs, sis, sos = (buf0, buf1), (si0, si1), (so0, so1)

        def in_dma(i, p):
            s = wid + i * NW
            return pltpu.make_async_copy(
                x_hbm.at[pl.ds(s * SLAB, SLAB)], bufs[p], sis[p])

        def out_dma(i, r, p):
            s = wid + i * NW
            b, ct = s >> 5, s & 31
            off = ((b * R + r) * CT + ct) * OSLAB
            return pltpu.make_async_copy(
                bufs[p].at[pl.ds(r * TILE, OSLAB)],
                out_hbm.at[pl.ds(off, OSLAB)], sos[p])

        in_dma(0, 0).start()

        def step(i, p):
            in_dma(i, p).wait()
            for r in range(R):
                out_dma(i, r, p).start()

            @pl.when(i + 1 < SPW)
            def _():
                @pl.when(i >= 1)
                def _():
                    for r in range(R):
                        out_dma(i - 1, r, 1 - p).wait()

                in_dma(i + 1, 1 - p).start()

        def pair(k2, _):
            step(k2 * 2, 0)
            step(k2 * 2 + 1, 1)
            return 0

        lax.fori_loop(0, SPW // 2, pair, 0)
        for r in range(R):
            out_dma(SPW - 2, r, 0).wait()
        for r in range(R):
            out_dma(SPW - 1, r, 1).wait()

    return k(x_lin)


def kernel(x):
    x_lin = x.reshape(B, CT, 8, TT, 128).transpose(0, 1, 3, 2, 4).reshape(-1)
    out_lin = _sc_chunk(x_lin)
    out = (out_lin.reshape(B, R, CT, 2, 8, 128)
           .transpose(0, 2, 4, 3, 5, 1)
           .reshape(16, 256, 256, 31))
    return out
